# baseline (speedup 1.0000x reference)
import functools

import jax
import jax.numpy as jnp
from jax import lax
from jax.experimental import pallas as pl
from jax.experimental.pallas import tpu as pltpu
from jax.experimental.pallas import tpu_sc as plsc

NC = 2
NS = 16
NW = NC * NS
K = 128

NBUF = 3


def _sc_aggregate(feat, src, dst, zeros, *, n, d, ew):
    n_acc = zeros.shape[0] * NS
    rows_z = zeros.shape[0]
    rows_out = 1000
    n_tiles_out = n // rows_out
    ch = ew // K

    mesh = plsc.VectorSubcoreMesh(core_axis_name="c", subcore_axis_name="s")

    @functools.partial(
        pl.kernel,
        out_type=jax.ShapeDtypeStruct((NC * n, d), jnp.float32),
        mesh=mesh,
        scratch_types=[
            pltpu.VMEM_SHARED((n_acc, d), jnp.float32),
            [pltpu.VMEM((K,), jnp.int32)] * NBUF,
            [pltpu.VMEM((K,), jnp.int32)] * NBUF,
            [pltpu.VMEM((K, d), jnp.float32)] * NBUF,
            [pltpu.SemaphoreType.DMA] * NBUF,
            [pltpu.SemaphoreType.DMA] * NBUF,
            [pltpu.SemaphoreType.DMA] * NBUF,
            [pltpu.SemaphoreType.DMA] * NBUF,
        ],
    )
    def sc_kernel(feat_hbm, src_hbm, dst_hbm, zero_hbm, out_hbm,
                  acc, src_v, dst_v, rows_v, gsem, ssem, isem, jsem):
        c = lax.axis_index("c")
        s = lax.axis_index("s")
        wid = c * NS + s

        pltpu.sync_copy(zero_hbm, acc.at[pl.ds(s * rows_z, rows_z)])
        plsc.subcore_barrier()

        base = wid * ew

        for b in range(NBUF):
            pltpu.sync_copy(src_hbm.at[pl.ds(base + b * K, K)], src_v[b])
            pltpu.sync_copy(dst_hbm.at[pl.ds(base + b * K, K)], dst_v[b])
            pltpu.async_copy(feat_hbm.at[src_v[b]], rows_v[b], gsem[b])

        def step(t, _):
            j0 = t * NBUF
            for b in range(NBUF):
                pltpu.make_async_copy(feat_hbm.at[src_v[b]],
                                      rows_v[b], gsem[b]).wait()
                pltpu.async_copy(rows_v[b], acc.at[dst_v[b]], ssem[b],
                                 add=True)
            for b in range(NBUF):
                pltpu.make_async_copy(rows_v[b], acc.at[dst_v[b]],
                                      ssem[b]).wait()
                off = base + jnp.minimum(j0 + b + NBUF, ch - 1) * K
                pltpu.async_copy(src_hbm.at[pl.ds(off, K)], src_v[b],
                                 isem[b])
                pltpu.async_copy(dst_hbm.at[pl.ds(off, K)], dst_v[b],
                                 jsem[b])
            for b in range(NBUF):
                pltpu.make_async_copy(src_hbm.at[pl.ds(base, K)],
                                      src_v[b], isem[b]).wait()
                pltpu.make_async_copy(dst_hbm.at[pl.ds(base, K)],
                                      dst_v[b], jsem[b]).wait()
                pltpu.async_copy(feat_hbm.at[src_v[b]], rows_v[b], gsem[b])
            return ()

        lax.fori_loop(0, ch // NBUF, step, (), unroll=False)

        for b in range(NBUF):
            pltpu.make_async_copy(feat_hbm.at[src_v[b]],
                                  rows_v[b], gsem[b]).wait()

        plsc.subcore_barrier()

        @pl.when(s < n_tiles_out)
        def _copy_out():
            pltpu.sync_copy(acc.at[pl.ds(s * rows_out, rows_out)],
                            out_hbm.at[pl.ds(c * n + s * rows_out, rows_out)])

    return sc_kernel(feat, src, dst, zeros)


def _tc_linear(feat, hp, norm, w1, w2, b1, b2, *, n, d, blk):
    nb = n // blk

    def body(feat_ref, h0_ref, h1_ref, norm_ref, w1_ref, w2_ref,
             b1_ref, b2_ref, out_ref):
        ah = (h0_ref[...] + h1_ref[...]) / norm_ref[...]
        dn = (((1,), (1,)), ((), ()))
        out_ref[...] = (
            lax.dot_general(feat_ref[...], w1_ref[...], dn,
                            preferred_element_type=jnp.float32)
            + lax.dot_general(ah, w2_ref[...], dn,
                              preferred_element_type=jnp.float32)
            + b1_ref[...] + b2_ref[...])

    return pl.pallas_call(
        body,
        grid=(nb,),
        in_specs=[
            pl.BlockSpec((blk, d), lambda i: (i, 0)),
            pl.BlockSpec((blk, d), lambda i: (i, 0)),
            pl.BlockSpec((blk, d), lambda i: (i + nb, 0)),
            pl.BlockSpec((blk, 1), lambda i: (i, 0)),
            pl.BlockSpec((d, d), lambda i: (0, 0)),
            pl.BlockSpec((d, d), lambda i: (0, 0)),
            pl.BlockSpec((1, d), lambda i: (0, 0)),
            pl.BlockSpec((1, d), lambda i: (0, 0)),
        ],
        out_specs=pl.BlockSpec((blk, d), lambda i: (i, 0)),
        out_shape=jax.ShapeDtypeStruct((n, d), jnp.float32),
    )(feat, hp, hp, norm, w1, w2, b1, b2)


def kernel(feat, edge_index, in_norm, W1, b1, W2, b2):
    n, d = feat.shape
    e = edge_index.shape[1]

    ew = -(-e // (NW * K * NBUF)) * (K * NBUF)
    pad = NW * ew - e
    src = jnp.concatenate([edge_index[0],
                           jnp.zeros((pad,), jnp.int32)])
    dst = jnp.concatenate([edge_index[1],
                           jnp.full((pad,), n, jnp.int32)])

    rows_z = -(-(n + 1) // (NS * 8)) * 8
    zeros = jnp.zeros((rows_z, d), jnp.float32)

    hp = _sc_aggregate(feat, src, dst, zeros, n=n, d=d, ew=ew)
    return _tc_linear(feat, hp, in_norm[:, None], W1, W2,
                      b1[None, :], b2[None, :], n=n, d=d, blk=1000)


# speedup vs baseline: 1.9272x; 1.9272x over previous
import functools

import jax
import jax.numpy as jnp
from jax import lax
from jax.experimental import pallas as pl
from jax.experimental.pallas import tpu as pltpu
from jax.experimental.pallas import tpu_sc as plsc

NC = 2
NS = 16
NW = NC * NS
K = 128
NP = 2


def _sc_aggregate(fh, src, dst, zeros, *, n, d, ew):
    dh = d // NP
    n_acc = zeros.shape[0] * NS
    rows_z = zeros.shape[0]
    rows_out = 1000
    n_tiles_out = n // rows_out
    ch = ew // K

    mesh = plsc.VectorSubcoreMesh(core_axis_name="c", subcore_axis_name="s")

    @functools.partial(
        pl.kernel,
        out_type=jax.ShapeDtypeStruct((NP, NC * n, dh), jnp.float32),
        mesh=mesh,
        scratch_types=[
            pltpu.VMEM_SHARED((n_acc, dh), jnp.float32),
            pltpu.VMEM_SHARED((n_acc, dh), jnp.float32),
            pltpu.VMEM((K,), jnp.int32),
            pltpu.VMEM((K,), jnp.int32),
            pltpu.VMEM((K, dh), jnp.float32),
            pltpu.SemaphoreType.DMA,
        ],
    )
    def sc_kernel(fh_hbm, src_hbm, dst_hbm, zero_hbm, out_hbm,
                  feat_s, acc, src_v, dst_v, rows_v, sem):
        c = lax.axis_index("c")
        s = lax.axis_index("s")
        wid = c * NS + s
        base = wid * ew

        for p in range(NP):
            pltpu.sync_copy(fh_hbm.at[p, pl.ds(s * rows_z, rows_z)],
                            feat_s.at[pl.ds(s * rows_z, rows_z)])
            pltpu.sync_copy(zero_hbm, acc.at[pl.ds(s * rows_z, rows_z)])
            plsc.subcore_barrier()

            def chunk(j, _):
                off = base + j * K
                pltpu.sync_copy(src_hbm.at[pl.ds(off, K)], src_v)
                pltpu.sync_copy(dst_hbm.at[pl.ds(off, K)], dst_v)
                pltpu.async_copy(feat_s.at[src_v], rows_v, sem).wait()
                pltpu.sync_copy(rows_v, acc.at[dst_v], add=True)
                return ()

            lax.fori_loop(0, ch, chunk, (), unroll=False)

            plsc.subcore_barrier()

            @pl.when(s < n_tiles_out)
            def _copy_out():
                pltpu.sync_copy(
                    acc.at[pl.ds(s * rows_out, rows_out)],
                    out_hbm.at[p, pl.ds(c * n + s * rows_out, rows_out)])

            plsc.subcore_barrier()

    return sc_kernel(fh, src, dst, zeros)


def _tc_linear(feat, hp, norm, w1, w2, b1, b2, *, n, d, blk):
    nb = n // blk
    dh = d // NP

    def body(feat_ref, h00_ref, h01_ref, h10_ref, h11_ref, norm_ref,
             w1_ref, w2_ref, b1_ref, b2_ref, out_ref):
        ah = jnp.concatenate(
            [h00_ref[0] + h01_ref[0], h10_ref[0] + h11_ref[0]],
            axis=-1) / norm_ref[...]
        dn = (((1,), (1,)), ((), ()))
        out_ref[...] = (
            lax.dot_general(feat_ref[...], w1_ref[...], dn,
                            preferred_element_type=jnp.float32)
            + lax.dot_general(ah, w2_ref[...], dn,
                              preferred_element_type=jnp.float32)
            + b1_ref[...] + b2_ref[...])

    return pl.pallas_call(
        body,
        grid=(nb,),
        in_specs=[
            pl.BlockSpec((blk, d), lambda i: (i, 0)),
            pl.BlockSpec((1, blk, dh), lambda i: (0, i, 0)),
            pl.BlockSpec((1, blk, dh), lambda i: (0, i + nb, 0)),
            pl.BlockSpec((1, blk, dh), lambda i: (1, i, 0)),
            pl.BlockSpec((1, blk, dh), lambda i: (1, i + nb, 0)),
            pl.BlockSpec((blk, 1), lambda i: (i, 0)),
            pl.BlockSpec((d, d), lambda i: (0, 0)),
            pl.BlockSpec((d, d), lambda i: (0, 0)),
            pl.BlockSpec((1, d), lambda i: (0, 0)),
            pl.BlockSpec((1, d), lambda i: (0, 0)),
        ],
        out_specs=pl.BlockSpec((blk, d), lambda i: (i, 0)),
        out_shape=jax.ShapeDtypeStruct((n, d), jnp.float32),
    )(feat, hp, hp, hp, hp, norm, w1, w2, b1, b2)


def kernel(feat, edge_index, in_norm, W1, b1, W2, b2):
    n, d = feat.shape
    e = edge_index.shape[1]
    dh = d // NP

    ew = -(-e // (NW * K)) * K
    pad = NW * ew - e
    src = jnp.concatenate([edge_index[0],
                           jnp.zeros((pad,), jnp.int32)])
    dst = jnp.concatenate([edge_index[1],
                           jnp.full((pad,), n, jnp.int32)])

    rows_z = -(-(n + 1) // (NS * 8)) * 8
    n_pad = rows_z * NS

    fh = jnp.stack([feat[:, p * dh:(p + 1) * dh] for p in range(NP)])
    fh = jnp.concatenate(
        [fh, jnp.zeros((NP, n_pad - n, dh), jnp.float32)], axis=1)
    zeros = jnp.zeros((rows_z, dh), jnp.float32)

    hp = _sc_aggregate(fh, src, dst, zeros, n=n, d=d, ew=ew)
    return _tc_linear(feat, hp, in_norm[:, None], W1, W2,
                      b1[None, :], b2[None, :], n=n, d=d, blk=1000)


# speedup vs baseline: 2.5699x; 1.3335x over previous
import functools

import jax
import jax.numpy as jnp
from jax import lax
from jax.experimental import pallas as pl
from jax.experimental.pallas import tpu as pltpu
from jax.experimental.pallas import tpu_sc as plsc

NC = 2
NS = 16
NW = NC * NS
K = 128
NP = 2


def _sc_aggregate(fh, src, dst, zeros, *, n, d, ew):
    dh = d // NP
    n_acc = zeros.shape[0] * NS
    rows_z = zeros.shape[0]
    rows_out = 1000
    n_tiles_out = n // rows_out
    ch = ew // K

    mesh = plsc.VectorSubcoreMesh(core_axis_name="c", subcore_axis_name="s")

    @functools.partial(
        pl.kernel,
        out_type=jax.ShapeDtypeStruct((NP, NC * n, dh), jnp.float32),
        mesh=mesh,
        scratch_types=[
            pltpu.VMEM_SHARED((n_acc, dh), jnp.float32),
            pltpu.VMEM_SHARED((n_acc, dh), jnp.float32),
            [pltpu.VMEM((K,), jnp.int32)] * 2,
            [pltpu.VMEM((K,), jnp.int32)] * 2,
            pltpu.VMEM((K, dh), jnp.float32),
            pltpu.SemaphoreType.DMA,
            [pltpu.SemaphoreType.DMA] * 2,
            [pltpu.SemaphoreType.DMA] * 2,
        ],
    )
    def sc_kernel(fh_hbm, src_hbm, dst_hbm, zero_hbm, out_hbm,
                  feat_s, acc, src_v, dst_v, rows_v, sem, isem, jsem):
        c = lax.axis_index("c")
        s = lax.axis_index("s")
        wid = c * NS + s
        base = wid * ew

        for p in range(NP):
            pltpu.sync_copy(fh_hbm.at[p, pl.ds(s * rows_z, rows_z)],
                            feat_s.at[pl.ds(s * rows_z, rows_z)])
            pltpu.sync_copy(zero_hbm, acc.at[pl.ds(s * rows_z, rows_z)])
            plsc.subcore_barrier()

            pltpu.sync_copy(src_hbm.at[pl.ds(base, K)], src_v[0])
            pltpu.sync_copy(dst_hbm.at[pl.ds(base, K)], dst_v[0])

            def pair(t, _):
                for b in range(2):
                    j = 2 * t + b
                    nxt = base + jnp.minimum(j + 1, ch - 1) * K
                    pltpu.async_copy(src_hbm.at[pl.ds(nxt, K)],
                                     src_v[1 - b], isem[1 - b])
                    pltpu.async_copy(dst_hbm.at[pl.ds(nxt, K)],
                                     dst_v[1 - b], jsem[1 - b])
                    pltpu.async_copy(feat_s.at[src_v[b]], rows_v,
                                     sem).wait()
                    pltpu.sync_copy(rows_v, acc.at[dst_v[b]], add=True)
                    pltpu.make_async_copy(src_hbm.at[pl.ds(base, K)],
                                          src_v[1 - b], isem[1 - b]).wait()
                    pltpu.make_async_copy(dst_hbm.at[pl.ds(base, K)],
                                          dst_v[1 - b], jsem[1 - b]).wait()
                return ()

            lax.fori_loop(0, ch // 2, pair, (), unroll=False)

            plsc.subcore_barrier()

            @pl.when(s < n_tiles_out)
            def _copy_out():
                pltpu.sync_copy(
                    acc.at[pl.ds(s * rows_out, rows_out)],
                    out_hbm.at[p, pl.ds(c * n + s * rows_out, rows_out)])

            plsc.subcore_barrier()

    return sc_kernel(fh, src, dst, zeros)


def _tc_linear(feat, hp, norm, w1, w2, b1, b2, *, n, d, blk):
    nb = n // blk
    dh = d // NP

    def body(feat_ref, h00_ref, h01_ref, h10_ref, h11_ref, norm_ref,
             w1_ref, w2_ref, b1_ref, b2_ref, out_ref):
        ah = jnp.concatenate(
            [h00_ref[0] + h01_ref[0], h10_ref[0] + h11_ref[0]],
            axis=-1) / norm_ref[...]
        dn = (((1,), (1,)), ((), ()))
        out_ref[...] = (
            lax.dot_general(feat_ref[...], w1_ref[...], dn,
                            preferred_element_type=jnp.float32)
            + lax.dot_general(ah, w2_ref[...], dn,
                              preferred_element_type=jnp.float32)
            + b1_ref[...] + b2_ref[...])

    return pl.pallas_call(
        body,
        grid=(nb,),
        in_specs=[
            pl.BlockSpec((blk, d), lambda i: (i, 0)),
            pl.BlockSpec((1, blk, dh), lambda i: (0, i, 0)),
            pl.BlockSpec((1, blk, dh), lambda i: (0, i + nb, 0)),
            pl.BlockSpec((1, blk, dh), lambda i: (1, i, 0)),
            pl.BlockSpec((1, blk, dh), lambda i: (1, i + nb, 0)),
            pl.BlockSpec((blk, 1), lambda i: (i, 0)),
            pl.BlockSpec((d, d), lambda i: (0, 0)),
            pl.BlockSpec((d, d), lambda i: (0, 0)),
            pl.BlockSpec((1, d), lambda i: (0, 0)),
            pl.BlockSpec((1, d), lambda i: (0, 0)),
        ],
        out_specs=pl.BlockSpec((blk, d), lambda i: (i, 0)),
        out_shape=jax.ShapeDtypeStruct((n, d), jnp.float32),
    )(feat, hp, hp, hp, hp, norm, w1, w2, b1, b2)


def kernel(feat, edge_index, in_norm, W1, b1, W2, b2):
    n, d = feat.shape
    e = edge_index.shape[1]
    dh = d // NP

    ew = -(-e // (NW * K)) * K
    pad = NW * ew - e
    src = jnp.concatenate([edge_index[0],
                           jnp.zeros((pad,), jnp.int32)])
    dst = jnp.concatenate([edge_index[1],
                           jnp.full((pad,), n, jnp.int32)])

    rows_z = -(-(n + 1) // (NS * 8)) * 8
    n_pad = rows_z * NS

    fh = jnp.stack([feat[:, p * dh:(p + 1) * dh] for p in range(NP)])
    fh = jnp.concatenate(
        [fh, jnp.zeros((NP, n_pad - n, dh), jnp.float32)], axis=1)
    zeros = jnp.zeros((rows_z, dh), jnp.float32)

    hp = _sc_aggregate(fh, src, dst, zeros, n=n, d=d, ew=ew)
    return _tc_linear(feat, hp, in_norm[:, None], W1, W2,
                      b1[None, :], b2[None, :], n=n, d=d, blk=1000)


# speedup vs baseline: 2.6163x; 1.0180x over previous
import functools

import jax
import jax.numpy as jnp
from jax import lax
from jax.experimental import pallas as pl
from jax.experimental.pallas import tpu as pltpu
from jax.experimental.pallas import tpu_sc as plsc

NC = 2
NS = 16
NW = NC * NS
K = 128
NP = 2


def _sc_aggregate(fh, src, dst, zeros, *, n, d, ew):
    dh = d // NP
    n_acc = zeros.shape[0] * NS
    rows_z = zeros.shape[0]
    rows_out = 1000
    n_tiles_out = n // rows_out
    ch = ew // K

    mesh = plsc.VectorSubcoreMesh(core_axis_name="c", subcore_axis_name="s")

    @functools.partial(
        pl.kernel,
        out_type=jax.ShapeDtypeStruct((NP, NC * n, dh), jnp.float32),
        mesh=mesh,
        scratch_types=[
            pltpu.VMEM_SHARED((n_acc, dh), jnp.float32),
            pltpu.VMEM_SHARED((n_acc, dh), jnp.float32),
            [pltpu.VMEM((K,), jnp.int32)] * 2,
            [pltpu.VMEM((K,), jnp.int32)] * 2,
            [pltpu.VMEM((K, dh), jnp.float32)] * 2,
            [pltpu.SemaphoreType.DMA] * 2,
            [pltpu.SemaphoreType.DMA] * 2,
            [pltpu.SemaphoreType.DMA] * 2,
        ],
    )
    def sc_kernel(fh_hbm, src_hbm, dst_hbm, zero_hbm, out_hbm,
                  feat_s, acc, src_v, dst_v, rows_v, gsem, isem, jsem):
        c = lax.axis_index("c")
        s = lax.axis_index("s")
        wid = c * NS + s
        base = wid * ew

        for p in range(NP):
            pltpu.sync_copy(fh_hbm.at[p, pl.ds(s * rows_z, rows_z)],
                            feat_s.at[pl.ds(s * rows_z, rows_z)])
            pltpu.sync_copy(zero_hbm, acc.at[pl.ds(s * rows_z, rows_z)])
            plsc.subcore_barrier()

            pltpu.sync_copy(src_hbm.at[pl.ds(base, K)], src_v[0])
            pltpu.sync_copy(dst_hbm.at[pl.ds(base, K)], dst_v[0])
            pltpu.async_copy(src_hbm.at[pl.ds(base + K, K)], src_v[1],
                             isem[1])
            pltpu.async_copy(dst_hbm.at[pl.ds(base + K, K)], dst_v[1],
                             jsem[1])
            pltpu.async_copy(feat_s.at[src_v[0]], rows_v[0], gsem[0])

            def pair(t, _):
                for b in range(2):
                    j = 2 * t + b
                    pltpu.make_async_copy(src_hbm.at[pl.ds(base, K)],
                                          src_v[1 - b], isem[1 - b]).wait()
                    pltpu.make_async_copy(dst_hbm.at[pl.ds(base, K)],
                                          dst_v[1 - b], jsem[1 - b]).wait()
                    pltpu.async_copy(feat_s.at[src_v[1 - b]],
                                     rows_v[1 - b], gsem[1 - b])
                    pltpu.make_async_copy(feat_s.at[src_v[b]],
                                          rows_v[b], gsem[b]).wait()
                    pltpu.sync_copy(rows_v[b], acc.at[dst_v[b]], add=True)
                    nxt = base + jnp.minimum(j + 2, ch - 1) * K
                    pltpu.async_copy(src_hbm.at[pl.ds(nxt, K)],
                                     src_v[b], isem[b])
                    pltpu.async_copy(dst_hbm.at[pl.ds(nxt, K)],
                                     dst_v[b], jsem[b])
                return ()

            lax.fori_loop(0, ch // 2, pair, (), unroll=False)

            pltpu.make_async_copy(feat_s.at[src_v[0]], rows_v[0],
                                  gsem[0]).wait()
            pltpu.make_async_copy(src_hbm.at[pl.ds(base, K)],
                                  src_v[1], isem[1]).wait()
            pltpu.make_async_copy(dst_hbm.at[pl.ds(base, K)],
                                  dst_v[1], jsem[1]).wait()

            plsc.subcore_barrier()

            @pl.when(s < n_tiles_out)
            def _copy_out():
                pltpu.sync_copy(
                    acc.at[pl.ds(s * rows_out, rows_out)],
                    out_hbm.at[p, pl.ds(c * n + s * rows_out, rows_out)])

            plsc.subcore_barrier()

    return sc_kernel(fh, src, dst, zeros)


def _tc_linear(feat, hp, norm, w1, w2, b1, b2, *, n, d, blk):
    nb = n // blk
    dh = d // NP

    def body(feat_ref, h00_ref, h01_ref, h10_ref, h11_ref, norm_ref,
             w1_ref, w2_ref, b1_ref, b2_ref, out_ref):
        ah = jnp.concatenate(
            [h00_ref[0] + h01_ref[0], h10_ref[0] + h11_ref[0]],
            axis=-1) / norm_ref[...]
        dn = (((1,), (1,)), ((), ()))
        out_ref[...] = (
            lax.dot_general(feat_ref[...], w1_ref[...], dn,
                            preferred_element_type=jnp.float32)
            + lax.dot_general(ah, w2_ref[...], dn,
                              preferred_element_type=jnp.float32)
            + b1_ref[...] + b2_ref[...])

    return pl.pallas_call(
        body,
        grid=(nb,),
        in_specs=[
            pl.BlockSpec((blk, d), lambda i: (i, 0)),
            pl.BlockSpec((1, blk, dh), lambda i: (0, i, 0)),
            pl.BlockSpec((1, blk, dh), lambda i: (0, i + nb, 0)),
            pl.BlockSpec((1, blk, dh), lambda i: (1, i, 0)),
            pl.BlockSpec((1, blk, dh), lambda i: (1, i + nb, 0)),
            pl.BlockSpec((blk, 1), lambda i: (i, 0)),
            pl.BlockSpec((d, d), lambda i: (0, 0)),
            pl.BlockSpec((d, d), lambda i: (0, 0)),
            pl.BlockSpec((1, d), lambda i: (0, 0)),
            pl.BlockSpec((1, d), lambda i: (0, 0)),
        ],
        out_specs=pl.BlockSpec((blk, d), lambda i: (i, 0)),
        out_shape=jax.ShapeDtypeStruct((n, d), jnp.float32),
    )(feat, hp, hp, hp, hp, norm, w1, w2, b1, b2)


def kernel(feat, edge_index, in_norm, W1, b1, W2, b2):
    n, d = feat.shape
    e = edge_index.shape[1]
    dh = d // NP

    ew = -(-e // (NW * K)) * K
    pad = NW * ew - e
    src = jnp.concatenate([edge_index[0],
                           jnp.zeros((pad,), jnp.int32)])
    dst = jnp.concatenate([edge_index[1],
                           jnp.full((pad,), n, jnp.int32)])

    rows_z = -(-(n + 1) // (NS * 8)) * 8
    n_pad = rows_z * NS

    fh = jnp.stack([feat[:, p * dh:(p + 1) * dh] for p in range(NP)])
    fh = jnp.concatenate(
        [fh, jnp.zeros((NP, n_pad - n, dh), jnp.float32)], axis=1)
    zeros = jnp.zeros((rows_z, dh), jnp.float32)

    hp = _sc_aggregate(fh, src, dst, zeros, n=n, d=d, ew=ew)
    return _tc_linear(feat, hp, in_norm[:, None], W1, W2,
                      b1[None, :], b2[None, :], n=n, d=d, blk=1000)
